# baseline (speedup 1.0000x reference)
import jax
import jax.numpy as jnp
from jax.experimental import pallas as pl
from jax.experimental.pallas import tpu as pltpu

E = 16
K = 2
D = 1024
F = 1024
T = 2048


NCH = 4


def _dense_kernel(x_ref, w_ref, wg_ref, wu_ref, wd_ref, out_ref):
    i = pl.program_id(0)
    x = x_ref[...]
    C = F // NCH
    y = None
    for c in range(NCH):
        wg = wg_ref[0, :, c * C:(c + 1) * C].astype(jnp.bfloat16)
        wu = wu_ref[0, :, c * C:(c + 1) * C].astype(jnp.bfloat16)
        g = jnp.dot(x, wg, preferred_element_type=jnp.float32)
        u = jnp.dot(x, wu, preferred_element_type=jnp.float32)
        h = (jax.nn.silu(g) * u).astype(jnp.bfloat16)
        wd = wd_ref[0, c * C:(c + 1) * C, :].astype(jnp.bfloat16)
        yc = jnp.dot(h, wd, preferred_element_type=jnp.float32)
        y = yc if y is None else y + yc
    y = y * w_ref[0, 0][:, None]

    @pl.when(i == 0)
    def _():
        out_ref[...] = y

    @pl.when(i > 0)
    def _():
        out_ref[...] += y


def kernel(hidden_states, gate_w, w_gate, w_up, w_down):
    logits = hidden_states @ gate_w
    probs = jax.nn.softmax(logits, axis=-1)
    i1 = jnp.argmax(probs, axis=-1).astype(jnp.int32)
    m1 = jnp.max(probs, axis=-1)
    eids = jnp.arange(E, dtype=jnp.int32)
    masked = jnp.where(eids[None, :] == i1[:, None], -1.0, probs)
    i2 = jnp.argmax(masked, axis=-1).astype(jnp.int32)
    m2 = jnp.max(masked, axis=-1)
    s = m1 + m2
    w_all = ((eids[:, None] == i1[None, :]) * (m1 / s)[None, :]
             + (eids[:, None] == i2[None, :]) * (m2 / s)[None, :])

    x = hidden_states.astype(jnp.bfloat16)

    out = pl.pallas_call(
        _dense_kernel,
        grid=(E,),
        in_specs=[
            pl.BlockSpec((T, D), lambda i: (0, 0)),
            pl.BlockSpec((1, 1, T), lambda i: (i, 0, 0)),
            pl.BlockSpec((1, D, F), lambda i: (i, 0, 0)),
            pl.BlockSpec((1, D, F), lambda i: (i, 0, 0)),
            pl.BlockSpec((1, F, D), lambda i: (i, 0, 0)),
        ],
        out_specs=pl.BlockSpec((T, D), lambda i: (0, 0)),
        out_shape=jax.ShapeDtypeStruct((T, D), jnp.float32),
        compiler_params=pltpu.CompilerParams(
            dimension_semantics=("arbitrary",),
            vmem_limit_bytes=64 * 1024 * 1024,
        ),
    )(x, w_all.reshape(E, 1, T), w_gate, w_up, w_down)
    return out
